# Initial kernel scaffold; baseline (speedup 1.0000x reference)
import functools

import jax
import jax.numpy as jnp
from jax import lax
from jax.experimental import pallas as pl
from jax.experimental.pallas import tpu as pltpu
from jax.experimental.pallas import tpu_sc as plsc

N = 10000
E = 320000
D = 128
G = 64

NC = 2
NS = 16
NW = NC * NS
EPW = E // NW
WIN = 100
NWIN = EPW // WIN
RPT = N // NS
ZR = 125

RB = 1000
NRB = N // RB


def _sc_gather_segsum(x, src_r, dst_r):
    mesh = plsc.VectorSubcoreMesh(core_axis_name="c", subcore_axis_name="s")

    @functools.partial(
        pl.kernel,
        out_type=jax.ShapeDtypeStruct((NC, N, D), jnp.float32),
        mesh=mesh,
        scratch_types=[
            pltpu.VMEM((NWIN, WIN), jnp.int32),
            pltpu.VMEM((NWIN, WIN), jnp.int32),
            pltpu.VMEM((WIN, D), jnp.float32),
            pltpu.VMEM((WIN, D), jnp.float32),
            pltpu.VMEM((ZR, D), jnp.float32),
            pltpu.VMEM_SHARED((N, D), jnp.float32),
            pltpu.SemaphoreType.DMA,
            pltpu.SemaphoreType.DMA,
        ],
    )
    def k(x_hbm, src_hbm, dst_hbm, out_hbm,
          src_v, dst_v, rows0, rows1, zbuf, agg_sh, sem0, sem1):
        cid = lax.axis_index("c")
        sid = lax.axis_index("s")
        gwid = cid * NS + sid

        pltpu.sync_copy(src_hbm.at[gwid], src_v)
        pltpu.sync_copy(dst_hbm.at[gwid], dst_v)

        zv = jnp.zeros((16,), jnp.float32)

        @pl.loop(0, ZR)
        def _(r):
            for c in range(D // 16):
                zbuf[r, pl.ds(c * 16, 16)] = zv

        for t in range(RPT // ZR):
            pltpu.sync_copy(zbuf, agg_sh.at[pl.ds(sid * RPT + t * ZR, ZR)])
        plsc.subcore_barrier()

        pltpu.make_async_copy(x_hbm.at[src_v.at[0]], rows0, sem0).start()

        @pl.loop(0, NWIN, step=2)
        def _(j):
            pltpu.make_async_copy(x_hbm.at[src_v.at[j]], rows0, sem0).wait()
            pltpu.make_async_copy(x_hbm.at[src_v.at[j + 1]], rows1, sem1).start()
            pltpu.sync_copy(rows0, agg_sh.at[dst_v.at[j]], add=True)
            pltpu.make_async_copy(x_hbm.at[src_v.at[j + 1]], rows1, sem1).wait()

            @pl.when(j + 2 < NWIN)
            def _():
                pltpu.make_async_copy(
                    x_hbm.at[src_v.at[j + 2]], rows0, sem0).start()

            pltpu.sync_copy(rows1, agg_sh.at[dst_v.at[j + 1]], add=True)

        plsc.subcore_barrier()
        pltpu.sync_copy(agg_sh.at[pl.ds(sid * RPT, RPT)],
                        out_hbm.at[cid].at[pl.ds(sid * RPT, RPT)])

    return k(x, src_r, dst_r)


def _dot(a, b):
    return lax.dot_general(a, b, (((1,), (0,)), ((), ())),
                           preferred_element_type=jnp.float32)


def _mlp_body(agg_ref, h_ref, w1_ref, b1_ref, w2_ref, b2_ref):
    a = agg_ref[0] + agg_ref[1] + h_ref[...]
    z = jnp.maximum(_dot(a, w1_ref[...]) + b1_ref[...], 0.0)
    return jnp.maximum(_dot(z, w2_ref[...]) + b2_ref[...], 0.0)


def _tc_mlp(agg, h, w1, b1, w2, b2):
    def body(agg_ref, h_ref, w1_ref, b1_ref, w2_ref, b2_ref, out_ref):
        out_ref[...] = _mlp_body(agg_ref, h_ref, w1_ref, b1_ref, w2_ref, b2_ref)

    full = lambda *_: (0, 0)
    return pl.pallas_call(
        body,
        grid=(NRB,),
        in_specs=[
            pl.BlockSpec((NC, RB, D), lambda i: (0, i, 0)),
            pl.BlockSpec((RB, D), lambda i: (i, 0)),
            pl.BlockSpec((D, D), full),
            pl.BlockSpec((1, D), full),
            pl.BlockSpec((D, D), full),
            pl.BlockSpec((1, D), full),
        ],
        out_specs=pl.BlockSpec((RB, D), lambda i: (i, 0)),
        out_shape=jax.ShapeDtypeStruct((N, D), jnp.float32),
    )(agg, h, w1, b1, w2, b2)


def _tc_mlp_pool(agg, h, w1, b1, w2, b2, batch_r):
    def body(agg_ref, h_ref, w1_ref, b1_ref, w2_ref, b2_ref, batch_ref,
             out_ref, acc_ref, cnt_ref):
        i = pl.program_id(0)

        @pl.when(i == 0)
        def _():
            acc_ref[...] = jnp.zeros_like(acc_ref)
            cnt_ref[...] = jnp.zeros_like(cnt_ref)

        h2 = _mlp_body(agg_ref, h_ref, w1_ref, b1_ref, w2_ref, b2_ref)
        bb = batch_ref[0, 0, :]
        gids = lax.broadcasted_iota(jnp.int32, (RB, G), 1)
        onehot = (bb[:, None] == gids).astype(jnp.float32)
        acc_ref[...] += lax.dot_general(
            onehot, h2, (((0,), (0,)), ((), ())),
            preferred_element_type=jnp.float32)
        cnt_ref[...] += jnp.broadcast_to(
            jnp.sum(onehot, axis=0)[:, None], (G, D))

        @pl.when(i == NRB - 1)
        def _():
            out_ref[...] = acc_ref[...] / jnp.maximum(cnt_ref[...], 1.0)

    full = lambda *_: (0, 0)
    return pl.pallas_call(
        body,
        grid=(NRB,),
        in_specs=[
            pl.BlockSpec((NC, RB, D), lambda i: (0, i, 0)),
            pl.BlockSpec((RB, D), lambda i: (i, 0)),
            pl.BlockSpec((D, D), full),
            pl.BlockSpec((1, D), full),
            pl.BlockSpec((D, D), full),
            pl.BlockSpec((1, D), full),
            pl.BlockSpec((1, 1, RB), lambda i: (i, 0, 0)),
        ],
        out_specs=pl.BlockSpec((G, D), full),
        out_shape=jax.ShapeDtypeStruct((G, D), jnp.float32),
        scratch_shapes=[
            pltpu.VMEM((G, D), jnp.float32),
            pltpu.VMEM((G, D), jnp.float32),
        ],
    )(agg, h, w1, b1, w2, b2, batch_r)


def kernel(x, edge_index, batch, W1a, b1a, W2a, b2a, W1b, b1b, W2b, b2b):
    src_r = edge_index[0].reshape(NW, NWIN, WIN)
    dst_r = edge_index[1].reshape(NW, NWIN, WIN)
    batch_r = batch.reshape(NRB, 1, RB)

    agg1 = _sc_gather_segsum(x, src_r, dst_r)
    h1 = _tc_mlp(agg1, x, W1a, b1a.reshape(1, D), W2a, b2a.reshape(1, D))
    agg2 = _sc_gather_segsum(h1, src_r, dst_r)
    return _tc_mlp_pool(agg2, h1, W1b, b1b.reshape(1, D),
                        W2b, b2b.reshape(1, D), batch_r)


# speedup vs baseline: 5.8742x; 5.8742x over previous
import functools

import jax
import jax.numpy as jnp
from jax import lax
from jax.experimental import pallas as pl
from jax.experimental.pallas import tpu as pltpu
from jax.experimental.pallas import tpu_sc as plsc

N = 10000
E = 320000
D = 128
G = 64

NS = 16
EPW = E // NS
WIN = 100
NWIN = EPW // WIN
CHW = 50
NCH = NWIN // CHW
RPT = 624
TAIL = N - NS * RPT

RB = 1000
NRB = N // RB


def _sc_gather_segsum(x, src_r, dst_r):
    mesh = plsc.VectorSubcoreMesh(
        core_axis_name="c", subcore_axis_name="s", num_cores=1)

    @functools.partial(
        pl.kernel,
        out_type=jax.ShapeDtypeStruct((N, D), jnp.float32),
        mesh=mesh,
        compiler_params=pltpu.CompilerParams(use_tc_tiling_on_sc=False),
        scratch_types=[
            pltpu.VMEM((CHW, WIN), jnp.int32),
            pltpu.VMEM((CHW, WIN), jnp.int32),
            pltpu.VMEM((WIN, D), jnp.float32),
            pltpu.VMEM((WIN, D), jnp.float32),
            pltpu.VMEM_SHARED((N, D), jnp.float32),
            pltpu.SemaphoreType.DMA,
            pltpu.SemaphoreType.DMA,
        ],
    )
    def k(x_hbm, src_hbm, dst_hbm, out_hbm,
          src_v, dst_v, rows0, rows1, agg_sh, sem0, sem1):
        sid = lax.axis_index("s")

        zv = jnp.zeros((16,), jnp.float32)

        @pl.loop(0, WIN)
        def _(r):
            for c in range(D // 16):
                rows0[r, pl.ds(c * 16, 16)] = zv

        for t in range(RPT // WIN):
            pltpu.sync_copy(rows0, agg_sh.at[pl.ds(sid * RPT + t * WIN, WIN)])
        pltpu.sync_copy(rows0.at[pl.ds(0, RPT % WIN)],
                        agg_sh.at[pl.ds(sid * RPT + RPT - RPT % WIN,
                                        RPT % WIN)])

        @pl.when(sid == 0)
        def _():
            pltpu.sync_copy(rows0.at[pl.ds(0, TAIL)],
                            agg_sh.at[pl.ds(NS * RPT, TAIL)])

        plsc.subcore_barrier()

        @pl.loop(0, NCH)
        def _(ci):
            pltpu.sync_copy(src_hbm.at[sid].at[pl.ds(ci * CHW, CHW)], src_v)
            pltpu.sync_copy(dst_hbm.at[sid].at[pl.ds(ci * CHW, CHW)], dst_v)
            pltpu.make_async_copy(x_hbm.at[src_v.at[0]], rows0, sem0).start()

            @pl.loop(0, CHW, step=2)
            def _(j):
                pltpu.make_async_copy(
                    x_hbm.at[src_v.at[j]], rows0, sem0).wait()
                pltpu.make_async_copy(
                    x_hbm.at[src_v.at[j + 1]], rows1, sem1).start()
                pltpu.sync_copy(rows0, agg_sh.at[dst_v.at[j]], add=True)
                pltpu.make_async_copy(
                    x_hbm.at[src_v.at[j + 1]], rows1, sem1).wait()

                @pl.when(j + 2 < CHW)
                def _():
                    pltpu.make_async_copy(
                        x_hbm.at[src_v.at[j + 2]], rows0, sem0).start()

                pltpu.sync_copy(rows1, agg_sh.at[dst_v.at[j + 1]], add=True)

        plsc.subcore_barrier()
        pltpu.sync_copy(agg_sh.at[pl.ds(sid * RPT, RPT)],
                        out_hbm.at[pl.ds(sid * RPT, RPT)])

        @pl.when(sid == 0)
        def _():
            pltpu.sync_copy(agg_sh.at[pl.ds(NS * RPT, TAIL)],
                            out_hbm.at[pl.ds(NS * RPT, TAIL)])

    return k(x, src_r, dst_r)


def _dot(a, b):
    return lax.dot_general(a, b, (((1,), (0,)), ((), ())),
                           preferred_element_type=jnp.float32)


def _mlp_body(agg_ref, h_ref, w1_ref, b1_ref, w2_ref, b2_ref):
    a = agg_ref[...] + h_ref[...]
    z = jnp.maximum(_dot(a, w1_ref[...]) + b1_ref[...], 0.0)
    return jnp.maximum(_dot(z, w2_ref[...]) + b2_ref[...], 0.0)


def _tc_mlp(agg, h, w1, b1, w2, b2):
    def body(agg_ref, h_ref, w1_ref, b1_ref, w2_ref, b2_ref, out_ref):
        out_ref[...] = _mlp_body(agg_ref, h_ref, w1_ref, b1_ref, w2_ref,
                                 b2_ref)

    full = lambda *_: (0, 0)
    return pl.pallas_call(
        body,
        grid=(NRB,),
        in_specs=[
            pl.BlockSpec((RB, D), lambda i: (i, 0)),
            pl.BlockSpec((RB, D), lambda i: (i, 0)),
            pl.BlockSpec((D, D), full),
            pl.BlockSpec((1, D), full),
            pl.BlockSpec((D, D), full),
            pl.BlockSpec((1, D), full),
        ],
        out_specs=pl.BlockSpec((RB, D), lambda i: (i, 0)),
        out_shape=jax.ShapeDtypeStruct((N, D), jnp.float32),
    )(agg, h, w1, b1, w2, b2)


def _tc_pool(h, batch_r):
    def body(h_ref, batch_ref, out_ref, acc_ref, cnt_ref):
        i = pl.program_id(0)

        @pl.when(i == 0)
        def _():
            acc_ref[...] = jnp.zeros_like(acc_ref)
            cnt_ref[...] = jnp.zeros_like(cnt_ref)

        bb = batch_ref[0, 0, :]
        gids = lax.broadcasted_iota(jnp.int32, (RB, G), 1)
        onehot = (bb[:, None] == gids).astype(jnp.float32)
        acc_ref[...] += lax.dot_general(
            onehot, h_ref[...], (((0,), (0,)), ((), ())),
            preferred_element_type=jnp.float32)
        cnt_ref[...] += jnp.broadcast_to(
            jnp.sum(onehot, axis=0)[:, None], (G, D))

        @pl.when(i == NRB - 1)
        def _():
            out_ref[...] = acc_ref[...] / jnp.maximum(cnt_ref[...], 1.0)

    full = lambda *_: (0, 0)
    return pl.pallas_call(
        body,
        grid=(NRB,),
        in_specs=[
            pl.BlockSpec((RB, D), lambda i: (i, 0)),
            pl.BlockSpec((1, 1, RB), lambda i: (i, 0, 0)),
        ],
        out_specs=pl.BlockSpec((G, D), full),
        out_shape=jax.ShapeDtypeStruct((G, D), jnp.float32),
        scratch_shapes=[
            pltpu.VMEM((G, D), jnp.float32),
            pltpu.VMEM((G, D), jnp.float32),
        ],
    )(h, batch_r)


def kernel(x, edge_index, batch, W1a, b1a, W2a, b2a, W1b, b1b, W2b, b2b):
    src_r = edge_index[0].reshape(NS, NWIN, WIN)
    dst_r = edge_index[1].reshape(NS, NWIN, WIN)
    batch_r = batch.reshape(NRB, 1, RB)

    w1s = jnp.stack([W1a, W1b])
    b1s = jnp.stack([b1a.reshape(1, D), b1b.reshape(1, D)])
    w2s = jnp.stack([W2a, W2b])
    b2s = jnp.stack([b2a.reshape(1, D), b2b.reshape(1, D)])

    def step(h, ws):
        w1, b1, w2, b2 = ws
        agg = _sc_gather_segsum(h, src_r, dst_r)
        return _tc_mlp(agg, h, w1, b1, w2, b2), None

    h2, _ = lax.scan(step, x, (w1s, b1s, w2s, b2s))
    return _tc_pool(h2, batch_r)


# speedup vs baseline: 6.4207x; 1.0930x over previous
import functools

import jax
import jax.numpy as jnp
from jax import lax
from jax.experimental import pallas as pl
from jax.experimental.pallas import tpu as pltpu
from jax.experimental.pallas import tpu_sc as plsc

N = 10000
E = 320000
D = 128
G = 64

NS = 16
EPW = E // NS
WIN = 125
NWIN = EPW // WIN
CHW = 40
NCH = NWIN // CHW
RPT = 624
TAIL = N - NS * RPT

RB = 1000
NRB = N // RB


def _sc_gather_segsum(x, src_r, dst_r):
    mesh = plsc.VectorSubcoreMesh(
        core_axis_name="c", subcore_axis_name="s", num_cores=1)

    @functools.partial(
        pl.kernel,
        out_type=jax.ShapeDtypeStruct((N, D), jnp.float32),
        mesh=mesh,
        compiler_params=pltpu.CompilerParams(use_tc_tiling_on_sc=False),
        scratch_types=[
            pltpu.VMEM((CHW, WIN), jnp.int32),
            pltpu.VMEM((CHW, WIN), jnp.int32),
            pltpu.VMEM((WIN, D), jnp.float32),
            pltpu.VMEM((WIN, D), jnp.float32),
            pltpu.VMEM_SHARED((N, D), jnp.float32),
            pltpu.SemaphoreType.DMA,
            pltpu.SemaphoreType.DMA,
        ],
    )
    def k(x_hbm, src_hbm, dst_hbm, out_hbm,
          src_v, dst_v, rows0, rows1, agg_sh, sem0, sem1):
        sid = lax.axis_index("s")

        zv = jnp.zeros((16,), jnp.float32)

        @pl.loop(0, WIN)
        def _(r):
            for c in range(D // 16):
                rows0[r, pl.ds(c * 16, 16)] = zv

        for t in range(RPT // 104):
            pltpu.sync_copy(rows0.at[pl.ds(0, 104)],
                            agg_sh.at[pl.ds(sid * RPT + t * 104, 104)])

        @pl.when(sid == 0)
        def _():
            pltpu.sync_copy(rows0.at[pl.ds(0, TAIL)],
                            agg_sh.at[pl.ds(NS * RPT, TAIL)])

        plsc.subcore_barrier()

        @pl.loop(0, NCH)
        def _(ci):
            pltpu.sync_copy(src_hbm.at[sid].at[pl.ds(ci * CHW, CHW)], src_v)
            pltpu.sync_copy(dst_hbm.at[sid].at[pl.ds(ci * CHW, CHW)], dst_v)
            pltpu.make_async_copy(x_hbm.at[src_v.at[0]], rows0, sem0).start()

            @pl.loop(0, CHW, step=2)
            def _(j):
                pltpu.make_async_copy(
                    x_hbm.at[src_v.at[j]], rows0, sem0).wait()
                pltpu.make_async_copy(
                    x_hbm.at[src_v.at[j + 1]], rows1, sem1).start()
                pltpu.sync_copy(rows0, agg_sh.at[dst_v.at[j]], add=True)
                pltpu.make_async_copy(
                    x_hbm.at[src_v.at[j + 1]], rows1, sem1).wait()

                @pl.when(j + 2 < CHW)
                def _():
                    pltpu.make_async_copy(
                        x_hbm.at[src_v.at[j + 2]], rows0, sem0).start()

                pltpu.sync_copy(rows1, agg_sh.at[dst_v.at[j + 1]], add=True)

        plsc.subcore_barrier()
        pltpu.sync_copy(agg_sh.at[pl.ds(sid * RPT, RPT)],
                        out_hbm.at[pl.ds(sid * RPT, RPT)])

        @pl.when(sid == 0)
        def _():
            pltpu.sync_copy(agg_sh.at[pl.ds(NS * RPT, TAIL)],
                            out_hbm.at[pl.ds(NS * RPT, TAIL)])

    return k(x, src_r, dst_r)


def _dot(a, b):
    return lax.dot_general(a, b, (((1,), (0,)), ((), ())),
                           preferred_element_type=jnp.float32)


def _mlp_body(agg_ref, h_ref, w1_ref, b1_ref, w2_ref, b2_ref):
    a = agg_ref[...] + h_ref[...]
    z = jnp.maximum(_dot(a, w1_ref[...]) + b1_ref[...], 0.0)
    return jnp.maximum(_dot(z, w2_ref[...]) + b2_ref[...], 0.0)


def _tc_mlp(agg, h, w1, b1, w2, b2):
    def body(agg_ref, h_ref, w1_ref, b1_ref, w2_ref, b2_ref, out_ref):
        out_ref[...] = _mlp_body(agg_ref, h_ref, w1_ref, b1_ref, w2_ref,
                                 b2_ref)

    full = lambda *_: (0, 0)
    return pl.pallas_call(
        body,
        grid=(NRB,),
        in_specs=[
            pl.BlockSpec((RB, D), lambda i: (i, 0)),
            pl.BlockSpec((RB, D), lambda i: (i, 0)),
            pl.BlockSpec((D, D), full),
            pl.BlockSpec((1, D), full),
            pl.BlockSpec((D, D), full),
            pl.BlockSpec((1, D), full),
        ],
        out_specs=pl.BlockSpec((RB, D), lambda i: (i, 0)),
        out_shape=jax.ShapeDtypeStruct((N, D), jnp.float32),
    )(agg, h, w1, b1, w2, b2)


def _tc_pool(h, batch_r):
    def body(h_ref, batch_ref, out_ref, acc_ref, cnt_ref):
        i = pl.program_id(0)

        @pl.when(i == 0)
        def _():
            acc_ref[...] = jnp.zeros_like(acc_ref)
            cnt_ref[...] = jnp.zeros_like(cnt_ref)

        bb = batch_ref[0, 0, :]
        gids = lax.broadcasted_iota(jnp.int32, (RB, G), 1)
        onehot = (bb[:, None] == gids).astype(jnp.float32)
        acc_ref[...] += lax.dot_general(
            onehot, h_ref[...], (((0,), (0,)), ((), ())),
            preferred_element_type=jnp.float32)
        cnt_ref[...] += jnp.broadcast_to(
            jnp.sum(onehot, axis=0)[:, None], (G, D))

        @pl.when(i == NRB - 1)
        def _():
            out_ref[...] = acc_ref[...] / jnp.maximum(cnt_ref[...], 1.0)

    full = lambda *_: (0, 0)
    return pl.pallas_call(
        body,
        grid=(NRB,),
        in_specs=[
            pl.BlockSpec((RB, D), lambda i: (i, 0)),
            pl.BlockSpec((1, 1, RB), lambda i: (i, 0, 0)),
        ],
        out_specs=pl.BlockSpec((G, D), full),
        out_shape=jax.ShapeDtypeStruct((G, D), jnp.float32),
        scratch_shapes=[
            pltpu.VMEM((G, D), jnp.float32),
            pltpu.VMEM((G, D), jnp.float32),
        ],
    )(h, batch_r)


def kernel(x, edge_index, batch, W1a, b1a, W2a, b2a, W1b, b1b, W2b, b2b):
    src_r = edge_index[0].reshape(NS, NWIN, WIN)
    dst_r = edge_index[1].reshape(NS, NWIN, WIN)
    batch_r = batch.reshape(NRB, 1, RB)

    w1s = jnp.stack([W1a, W1b])
    b1s = jnp.stack([b1a.reshape(1, D), b1b.reshape(1, D)])
    w2s = jnp.stack([W2a, W2b])
    b2s = jnp.stack([b2a.reshape(1, D), b2b.reshape(1, D)])

    def step(h, ws):
        w1, b1, w2, b2 = ws
        agg = _sc_gather_segsum(h, src_r, dst_r)
        return _tc_mlp(agg, h, w1, b1, w2, b2), None

    h2, _ = lax.scan(step, x, (w1s, b1s, w2s, b2s))
    return _tc_pool(h2, batch_r)


# speedup vs baseline: 7.2772x; 1.1334x over previous
import functools

import jax
import jax.numpy as jnp
from jax import lax
from jax.experimental import pallas as pl
from jax.experimental.pallas import tpu as pltpu
from jax.experimental.pallas import tpu_sc as plsc

N = 10000
E = 320000
D = 128
G = 64

NC = 2
NS = 16
DH = D // NC
EPW = E // NS
WIN = 125
NWIN = EPW // WIN
CHW = 20
NCH = NWIN // CHW
RPT = 624
TAIL = N - NS * RPT

RB = 1000
NRB = N // RB


def _sc_gather_segsum(xh, src_r, dst_r):
    mesh = plsc.VectorSubcoreMesh(
        core_axis_name="c", subcore_axis_name="s", num_cores=NC)

    @functools.partial(
        pl.kernel,
        out_type=jax.ShapeDtypeStruct((NC, N, DH), jnp.float32),
        mesh=mesh,
        compiler_params=pltpu.CompilerParams(use_tc_tiling_on_sc=False),
        scratch_types=[
            pltpu.VMEM((CHW, WIN), jnp.int32),
            pltpu.VMEM((CHW, WIN), jnp.int32),
            pltpu.VMEM((WIN, DH), jnp.float32),
            pltpu.VMEM((WIN, DH), jnp.float32),
            pltpu.VMEM_SHARED((N, DH), jnp.float32),
            pltpu.SemaphoreType.DMA,
            pltpu.SemaphoreType.DMA,
        ],
    )
    def k(x_hbm, src_hbm, dst_hbm, out_hbm,
          src_v, dst_v, rows0, rows1, agg_sh, sem0, sem1):
        cid = lax.axis_index("c")
        sid = lax.axis_index("s")
        xc = x_hbm.at[cid]

        zv = jnp.zeros((16,), jnp.float32)

        @pl.loop(0, WIN)
        def _(r):
            for c in range(DH // 16):
                rows0[r, pl.ds(c * 16, 16)] = zv

        for t in range(RPT // 104):
            pltpu.sync_copy(rows0.at[pl.ds(0, 104)],
                            agg_sh.at[pl.ds(sid * RPT + t * 104, 104)])

        @pl.when(sid == 0)
        def _():
            pltpu.sync_copy(rows0.at[pl.ds(0, TAIL)],
                            agg_sh.at[pl.ds(NS * RPT, TAIL)])

        plsc.subcore_barrier()

        @pl.loop(0, NCH)
        def _(ci):
            pltpu.sync_copy(src_hbm.at[sid].at[pl.ds(ci * CHW, CHW)], src_v)
            pltpu.sync_copy(dst_hbm.at[sid].at[pl.ds(ci * CHW, CHW)], dst_v)
            pltpu.make_async_copy(xc.at[src_v.at[0]], rows0, sem0).start()

            @pl.loop(0, CHW, step=2)
            def _(j):
                pltpu.make_async_copy(xc.at[src_v.at[j]], rows0, sem0).wait()
                pltpu.make_async_copy(
                    xc.at[src_v.at[j + 1]], rows1, sem1).start()
                pltpu.sync_copy(rows0, agg_sh.at[dst_v.at[j]], add=True)
                pltpu.make_async_copy(
                    xc.at[src_v.at[j + 1]], rows1, sem1).wait()

                @pl.when(j + 2 < CHW)
                def _():
                    pltpu.make_async_copy(
                        xc.at[src_v.at[j + 2]], rows0, sem0).start()

                pltpu.sync_copy(rows1, agg_sh.at[dst_v.at[j + 1]], add=True)

        plsc.subcore_barrier()
        pltpu.sync_copy(agg_sh.at[pl.ds(sid * RPT, RPT)],
                        out_hbm.at[cid].at[pl.ds(sid * RPT, RPT)])

        @pl.when(sid == 0)
        def _():
            pltpu.sync_copy(agg_sh.at[pl.ds(NS * RPT, TAIL)],
                            out_hbm.at[cid].at[pl.ds(NS * RPT, TAIL)])

    return k(xh, src_r, dst_r)


def _dot(a, b):
    return lax.dot_general(a, b, (((1,), (0,)), ((), ())),
                           preferred_element_type=jnp.float32)


def _mlp_body(agg_ref, h_ref, w1_ref, b1_ref, w2_ref, b2_ref):
    a = jnp.concatenate(
        [agg_ref[0] + h_ref[0], agg_ref[1] + h_ref[1]], axis=1)
    z = jnp.maximum(_dot(a, w1_ref[...]) + b1_ref[...], 0.0)
    return jnp.maximum(_dot(z, w2_ref[...]) + b2_ref[...], 0.0)


def _tc_mlp(agg, h, w1, b1, w2, b2):
    def body(agg_ref, h_ref, w1_ref, b1_ref, w2_ref, b2_ref, out_ref):
        h2 = _mlp_body(agg_ref, h_ref, w1_ref, b1_ref, w2_ref, b2_ref)
        out_ref[0] = h2[:, :DH]
        out_ref[1] = h2[:, DH:]

    full = lambda *_: (0, 0)
    return pl.pallas_call(
        body,
        grid=(NRB,),
        in_specs=[
            pl.BlockSpec((NC, RB, DH), lambda i: (0, i, 0)),
            pl.BlockSpec((NC, RB, DH), lambda i: (0, i, 0)),
            pl.BlockSpec((D, D), full),
            pl.BlockSpec((1, D), full),
            pl.BlockSpec((D, D), full),
            pl.BlockSpec((1, D), full),
        ],
        out_specs=pl.BlockSpec((NC, RB, DH), lambda i: (0, i, 0)),
        out_shape=jax.ShapeDtypeStruct((NC, N, DH), jnp.float32),
    )(agg, h, w1, b1, w2, b2)


def _tc_pool(h, batch_r):
    def body(h_ref, batch_ref, out_ref, acc_ref, cnt_ref):
        i = pl.program_id(0)

        @pl.when(i == 0)
        def _():
            acc_ref[...] = jnp.zeros_like(acc_ref)
            cnt_ref[...] = jnp.zeros_like(cnt_ref)

        hh = jnp.concatenate([h_ref[0], h_ref[1]], axis=1)
        bb = batch_ref[0, 0, :]
        gids = lax.broadcasted_iota(jnp.int32, (RB, G), 1)
        onehot = (bb[:, None] == gids).astype(jnp.float32)
        acc_ref[...] += lax.dot_general(
            onehot, hh, (((0,), (0,)), ((), ())),
            preferred_element_type=jnp.float32)
        cnt_ref[...] += jnp.broadcast_to(
            jnp.sum(onehot, axis=0)[:, None], (G, D))

        @pl.when(i == NRB - 1)
        def _():
            out_ref[...] = acc_ref[...] / jnp.maximum(cnt_ref[...], 1.0)

    full = lambda *_: (0, 0)
    return pl.pallas_call(
        body,
        grid=(NRB,),
        in_specs=[
            pl.BlockSpec((NC, RB, DH), lambda i: (0, i, 0)),
            pl.BlockSpec((1, 1, RB), lambda i: (i, 0, 0)),
        ],
        out_specs=pl.BlockSpec((G, D), full),
        out_shape=jax.ShapeDtypeStruct((G, D), jnp.float32),
        scratch_shapes=[
            pltpu.VMEM((G, D), jnp.float32),
            pltpu.VMEM((G, D), jnp.float32),
        ],
    )(h, batch_r)


def kernel(x, edge_index, batch, W1a, b1a, W2a, b2a, W1b, b1b, W2b, b2b):
    src_r = edge_index[0].reshape(NS, NWIN, WIN)
    dst_r = edge_index[1].reshape(NS, NWIN, WIN)
    batch_r = batch.reshape(NRB, 1, RB)
    xh = jnp.stack([x[:, :DH], x[:, DH:]])

    w1s = jnp.stack([W1a, W1b])
    b1s = jnp.stack([b1a.reshape(1, D), b1b.reshape(1, D)])
    w2s = jnp.stack([W2a, W2b])
    b2s = jnp.stack([b2a.reshape(1, D), b2b.reshape(1, D)])

    def step(h, ws):
        w1, b1, w2, b2 = ws
        agg = _sc_gather_segsum(h, src_r, dst_r)
        return _tc_mlp(agg, h, w1, b1, w2, b2), None

    h2, _ = lax.scan(step, xh, (w1s, b1s, w2s, b2s))
    return _tc_pool(h2, batch_r)
